# baseline (speedup 1.0000x reference)
import functools

import jax
import jax.numpy as jnp
from jax import lax
from jax.experimental import pallas as pl
from jax.experimental.pallas import tpu as pltpu
from jax.experimental.pallas import tpu_sc as plsc

ROWS, COLS = 16384, 200
VOCAB, DIM = 64, 64
B = ROWS * COLS
NW = 32
R_PER_W = ROWS // NW
G = 4
CHUNK = G * COLS
N_CHUNKS = R_PER_W // G
N_PAIRS = N_CHUNKS // 2


def _make_kernel():
    mesh = plsc.VectorSubcoreMesh(core_axis_name="c", subcore_axis_name="s")

    @functools.partial(
        pl.kernel,
        mesh=mesh,
        out_type=jax.ShapeDtypeStruct((B, DIM), jnp.float32),
        scratch_types=[
            pltpu.VMEM((G, COLS), jnp.int32),
            pltpu.VMEM((G, COLS), jnp.int32),
            pltpu.VMEM((CHUNK, DIM), jnp.float32),
            pltpu.VMEM((CHUNK, DIM), jnp.float32),
            pltpu.VMEM_SHARED((VOCAB, DIM), jnp.float32),
            pltpu.SemaphoreType.DMA,
            pltpu.SemaphoreType.DMA,
            pltpu.SemaphoreType.DMA,
            pltpu.SemaphoreType.DMA,
            pltpu.SemaphoreType.DMA,
            pltpu.SemaphoreType.DMA,
        ],
        compiler_params=pltpu.CompilerParams(use_tc_tiling_on_sc=False),
    )
    def gather_kernel(idx_hbm, table_hbm, out_hbm,
                      idx0, idx1, rows0, rows1, table_v,
                      si0, si1, sg0, sg1, so0, so1):
        wid = lax.axis_index("s") * 2 + lax.axis_index("c")
        w_row = wid * R_PER_W
        idx_v = (idx0, idx1)
        rows_v = (rows0, rows1)
        sem_i = (si0, si1)
        sem_g = (sg0, sg1)
        sem_o = (so0, so1)

        pltpu.sync_copy(table_hbm, table_v)

        for b in range(2):
            pltpu.async_copy(
                idx_hbm.at[pl.ds(w_row + b * G, G), :], idx_v[b], sem_i[b])

        def body(j, _):
            for b in range(2):
                row0 = w_row + (2 * j + b) * G
                pltpu.make_async_copy(
                    idx_hbm.at[pl.ds(w_row, G), :], idx_v[b],
                    sem_i[b]).wait()

                @pl.when(j >= 1)
                def _():
                    pltpu.make_async_copy(
                        rows_v[b], out_hbm.at[pl.ds(0, CHUNK)],
                        sem_o[b]).wait()

                for g in range(G):
                    pltpu.async_copy(
                        table_v.at[idx_v[b].at[g]],
                        rows_v[b].at[pl.ds(g * COLS, COLS)],
                        sem_g[b])
                for g in range(G):
                    pltpu.make_async_copy(
                        table_v.at[idx_v[b].at[g]],
                        rows_v[b].at[pl.ds(0, COLS)],
                        sem_g[b]).wait()

                @pl.when(j < N_PAIRS - 1)
                def _():
                    pltpu.async_copy(
                        idx_hbm.at[pl.ds(row0 + 2 * G, G), :],
                        idx_v[b], sem_i[b])

                pltpu.async_copy(
                    rows_v[b], out_hbm.at[pl.ds(row0 * COLS, CHUNK)],
                    sem_o[b])
            return 0

        lax.fori_loop(0, N_PAIRS, body, 0)

        for b in range(2):
            pltpu.make_async_copy(
                rows_v[b], out_hbm.at[pl.ds(0, CHUNK)], sem_o[b]).wait()

    return gather_kernel


_gather = _make_kernel()


@jax.jit
def kernel(X, table):
    flat = _gather(X, table)
    return flat.reshape(ROWS, COLS, DIM)
